# baseline (speedup 1.0000x reference)
import jax
import jax.numpy as jnp
from jax import lax
from jax.experimental import pallas as pl
from jax.experimental.pallas import tpu as pltpu
from jax.experimental.pallas import tpu_sc as plsc

N = 10000
E = 320000
IN = 128
HID = 256
FEAT = 16

NC = 2
NS = 16
NW = NC * NS

_MESH = plsc.VectorSubcoreMesh(core_axis_name="c", subcore_axis_name="s")


def _pack_pairs(lo_f32, hi_f32):
    lo = lax.bitcast_convert_type(lo_f32.astype(jnp.bfloat16), jnp.uint16)
    hi = lax.bitcast_convert_type(hi_f32.astype(jnp.bfloat16), jnp.uint16)
    word = lo.astype(jnp.uint32) | (hi.astype(jnp.uint32) << 16)
    return lax.bitcast_convert_type(word, jnp.int32)
_SC_PARAMS = pltpu.CompilerParams(needs_layout_passes=False)

_ETA = E // NW
_CHA = 80
_NCA = _ETA // _CHA
_RCH = 80
_NRC = N // _RCH


def _own_rows(sid, body):
    nch = jnp.where(sid < _NRC % NS, _NRC // NS + 1, _NRC // NS)

    def f(k, c):
        r0 = pl.multiple_of((sid + NS * k) * _RCH, 8)
        body(r0)
        return c
    lax.fori_loop(0, nch, f, 0)


_DW = 8


def _sc_agg0(src_h, dst_h, x_h, outp_h, outd_h,
             idx_s0, idx_d0, idx_s1, idx_d1, rows0, rows1, zb, dacc,
             semis0, semis1, semid0, semid1, semr0, semr1, semsc0, semsc1,
             acc):
    cc = lax.axis_index("c")
    sid = lax.axis_index("s")
    wid = sid * NC + cc
    zv = jnp.zeros((16,), jnp.float32)
    ov = jnp.ones((16,), jnp.float32)
    slots = [(idx_s0, idx_d0, rows0, semis0, semid0, semr0, semsc0),
             (idx_s1, idx_d1, rows1, semis1, semid1, semr1, semsc1)]

    def zrow(r, carry):
        for j in range(IN // 16):
            zb[r, pl.ds(j * 16, 16)] = zv
        return carry
    lax.fori_loop(0, _RCH, zrow, 0)

    def zdrow(r, carry):
        dacc[pl.ds(pl.multiple_of(r * 16, 16), 16)] = zv
        return carry
    lax.fori_loop(0, N // 16, zdrow, 0)

    _own_rows(sid, lambda r0: pltpu.sync_copy(zb, acc.at[pl.ds(r0, _RCH)]))
    plsc.subcore_barrier()

    base = cc * (E // NC) + sid * _ETA

    def issue_idx_s(k, sl):
        off = pl.multiple_of(base + k * _CHA, 8)
        pltpu.async_copy(src_h.at[pl.ds(off, _CHA)], sl[0], sl[3])

    def issue_idx_d(k, sl):
        off = pl.multiple_of(base + k * _CHA, 8)
        pltpu.async_copy(dst_h.at[pl.ds(off, _CHA)], sl[1], sl[4])

    def drain_idx_s(sl):
        pltpu.make_async_copy(src_h.at[pl.ds(0, _CHA)], sl[0], sl[3]).wait()

    def drain_idx_d(sl):
        pltpu.make_async_copy(dst_h.at[pl.ds(0, _CHA)], sl[1], sl[4]).wait()

    def issue_gather(sl):
        pltpu.async_copy(x_h.at[sl[0]], sl[2], sl[5])

    def drain_gather(sl):
        pltpu.make_async_copy(x_h.at[pl.ds(0, _CHA)], sl[2], sl[5]).wait()

    def issue_scat(sl):
        pltpu.async_copy(sl[2], acc.at[sl[1]], sl[6], add=True)

    def drain_scat(sl):
        pltpu.make_async_copy(sl[2], acc.at[sl[1]], sl[6]).wait()

    issue_idx_s(0, slots[0])
    issue_idx_s(1, slots[1])
    issue_idx_d(0, slots[0])
    drain_idx_s(slots[0])
    issue_gather(slots[0])

    def chunkpair(i, carry):
        kk = i * 2
        for b in range(2):
            me, ot = slots[b], slots[1 - b]
            k = kk + b

            @pl.when(k < _NCA)
            def _():
                @pl.when(k >= 1)
                def _():
                    drain_scat(ot)

                @pl.when(k + 1 < _NCA)
                def _():
                    drain_idx_s(ot)
                    issue_gather(ot)
                    issue_idx_d(k + 1, ot)
                drain_gather(me)
                drain_idx_d(me)
                issue_scat(me)

                def dcount(i2, c2):
                    iv = me[1][pl.ds(pl.multiple_of(i2 * 16, 16), 16)]
                    plsc.addupdate_scatter(dacc, [iv], ov)
                    return c2
                lax.fori_loop(0, _CHA // 16, dcount, 0)

                @pl.when(k + 2 < _NCA)
                def _():
                    issue_idx_s(k + 2, me)
        return carry
    lax.fori_loop(0, (_NCA + 1) // 2, chunkpair, 0)
    drain_scat(slots[0])
    plsc.subcore_barrier()

    def ochunk(r0):
        pltpu.sync_copy(acc.at[pl.ds(r0, _RCH)], zb)
        pltpu.sync_copy(zb, outp_h.at[cc, pl.ds(r0, _RCH)])
    _own_rows(sid, ochunk)
    pltpu.sync_copy(dacc, outd_h.at[wid])


_agg0_call = pl.kernel(
    _sc_agg0,
    out_type=(jax.ShapeDtypeStruct((NC, N, IN), jnp.float32),
              jax.ShapeDtypeStruct((NW, N), jnp.float32)),
    mesh=_MESH,
    compiler_params=_SC_PARAMS,
    scratch_types=[
        pltpu.VMEM((_CHA,), jnp.int32),
        pltpu.VMEM((_CHA,), jnp.int32),
        pltpu.VMEM((_CHA,), jnp.int32),
        pltpu.VMEM((_CHA,), jnp.int32),
        pltpu.VMEM((_CHA, IN), jnp.float32),
        pltpu.VMEM((_CHA, IN), jnp.float32),
        pltpu.VMEM((_RCH, IN), jnp.float32),
        pltpu.VMEM((N,), jnp.float32),
        pltpu.SemaphoreType.DMA,
        pltpu.SemaphoreType.DMA,
        pltpu.SemaphoreType.DMA,
        pltpu.SemaphoreType.DMA,
        pltpu.SemaphoreType.DMA,
        pltpu.SemaphoreType.DMA,
        pltpu.SemaphoreType.DMA,
        pltpu.SemaphoreType.DMA,
        pltpu.VMEM_SHARED((N, IN), jnp.float32),
    ],
)

_ETB = E // NS
_CHB = 80
_NCB = _ETB // _CHB


def _sc_agg1(src_h, dst_h, h1a_h, h1b_h, out_h,
             idx_s0, idx_d0, idx_s1, idx_d1, rows0, rows1, zb,
             semis0, semis1, semid0, semid1, semr0, semr1, semsc0, semsc1,
             acc):
    cc = lax.axis_index("c")
    sid = lax.axis_index("s")
    zv = jnp.zeros((16,), jnp.float32)
    slots = [(idx_s0, idx_d0, rows0, semis0, semid0, semr0, semsc0),
             (idx_s1, idx_d1, rows1, semis1, semid1, semr1, semsc1)]

    def zrow(r, carry):
        for j in range(IN // 16):
            zb[r, pl.ds(j * 16, 16)] = zv
        return carry
    lax.fori_loop(0, _RCH, zrow, 0)
    _own_rows(sid, lambda r0: pltpu.sync_copy(zb, acc.at[pl.ds(r0, _RCH)]))
    plsc.subcore_barrier()

    base = sid * _ETB

    def issue_idx_s(k, sl):
        off = pl.multiple_of(base + k * _CHB, 8)
        pltpu.async_copy(src_h.at[pl.ds(off, _CHB)], sl[0], sl[3])

    def issue_idx_d(k, sl):
        off = pl.multiple_of(base + k * _CHB, 8)
        pltpu.async_copy(dst_h.at[pl.ds(off, _CHB)], sl[1], sl[4])

    def drain_idx_s(sl):
        pltpu.make_async_copy(src_h.at[pl.ds(0, _CHB)], sl[0], sl[3]).wait()

    def drain_idx_d(sl):
        pltpu.make_async_copy(dst_h.at[pl.ds(0, _CHB)], sl[1], sl[4]).wait()

    def issue_gather(sl):
        @pl.when(cc == 0)
        def _():
            pltpu.async_copy(h1a_h.at[sl[0]], sl[2], sl[5])

        @pl.when(cc == 1)
        def _():
            pltpu.async_copy(h1b_h.at[sl[0]], sl[2], sl[5])

    def drain_gather(sl):
        pltpu.make_async_copy(h1a_h.at[pl.ds(0, _CHB)], sl[2], sl[5]).wait()

    def issue_scat(sl):
        pltpu.async_copy(sl[2], acc.at[sl[1]], sl[6], add=True)

    def drain_scat(sl):
        pltpu.make_async_copy(sl[2], acc.at[sl[1]], sl[6]).wait()

    issue_idx_s(0, slots[0])
    issue_idx_s(1, slots[1])
    issue_idx_d(0, slots[0])
    drain_idx_s(slots[0])
    issue_gather(slots[0])

    def chunkpair(i, carry):
        kk = i * 2
        for b in range(2):
            me, ot = slots[b], slots[1 - b]
            k = kk + b

            @pl.when(k >= 1)
            def _():
                drain_scat(ot)

            @pl.when(k + 1 < _NCB)
            def _():
                drain_idx_s(ot)
                issue_gather(ot)
                issue_idx_d(k + 1, ot)
            drain_gather(me)
            drain_idx_d(me)
            issue_scat(me)

            @pl.when(k + 2 < _NCB)
            def _():
                issue_idx_s(k + 2, me)
        return carry
    lax.fori_loop(0, _NCB // 2, chunkpair, 0)
    drain_scat(slots[1])
    plsc.subcore_barrier()

    def ochunk(r0):
        pltpu.sync_copy(acc.at[pl.ds(r0, _RCH)], zb)
        pltpu.sync_copy(zb, out_h.at[cc, pl.ds(r0, _RCH)])
    _own_rows(sid, ochunk)


_agg1_call = pl.kernel(
    _sc_agg1,
    out_type=jax.ShapeDtypeStruct((NC, N, IN), jnp.float32),
    mesh=_MESH,
    compiler_params=_SC_PARAMS,
    scratch_types=[
        pltpu.VMEM((_CHB,), jnp.int32),
        pltpu.VMEM((_CHB,), jnp.int32),
        pltpu.VMEM((_CHB,), jnp.int32),
        pltpu.VMEM((_CHB,), jnp.int32),
        pltpu.VMEM((_CHB, IN), jnp.float32),
        pltpu.VMEM((_CHB, IN), jnp.float32),
        pltpu.VMEM((_RCH, IN), jnp.float32),
        pltpu.SemaphoreType.DMA,
        pltpu.SemaphoreType.DMA,
        pltpu.SemaphoreType.DMA,
        pltpu.SemaphoreType.DMA,
        pltpu.SemaphoreType.DMA,
        pltpu.SemaphoreType.DMA,
        pltpu.SemaphoreType.DMA,
        pltpu.SemaphoreType.DMA,
        pltpu.VMEM_SHARED((N, IN), jnp.float32),
    ],
)

_ETC = E // NW
_CHC = 80
_NCC = _ETC // _CHC


def _sc_cls(src_h, dst_h, a_h, b_h, g_h,
            idx_s0, idx_d0, idx_s1, idx_d1,
            rowsa0, rowsb0, outg0, rowsa1, rowsb1, outg1,
            semi0, semi1, semg0, semg1, semo0, semo1):
    cc = lax.axis_index("c")
    sid = lax.axis_index("s")
    wid = sid * NC + cc
    base = wid * _ETC
    slots = [(idx_s0, idx_d0, rowsa0, rowsb0, outg0, semi0, semg0, semo0),
             (idx_s1, idx_d1, rowsa1, rowsb1, outg1, semi1, semg1, semo1)]

    def issue_idx(k, sl):
        off = pl.multiple_of(base + k * _CHC, 8)
        pltpu.async_copy(src_h.at[pl.ds(off, _CHC)], sl[0], sl[5])
        pltpu.async_copy(dst_h.at[pl.ds(off, _CHC)], sl[1], sl[5])

    def drain_idx(sl):
        pltpu.make_async_copy(src_h.at[pl.ds(0, _CHC)], sl[0], sl[5]).wait()
        pltpu.make_async_copy(src_h.at[pl.ds(0, _CHC)], sl[1], sl[5]).wait()

    def issue_gathers(sl):
        pltpu.async_copy(a_h.at[sl[0]], sl[2], sl[6])
        pltpu.async_copy(b_h.at[sl[1]], sl[3], sl[6])

    def drain_gathers(sl):
        for r in (sl[2], sl[3]):
            pltpu.make_async_copy(a_h.at[pl.ds(0, _CHC)], r, sl[6]).wait()

    issue_idx(0, slots[0])
    issue_idx(1, slots[1])
    drain_idx(slots[0])
    issue_gathers(slots[0])

    def chunkpair(i, carry):
        kk = i * 2
        for b in range(2):
            me, ot = slots[b], slots[1 - b]
            k = kk + b

            @pl.when(k < _NCC)
            def _():
                @pl.when(k + 1 < _NCC)
                def _():
                    drain_idx(ot)
                    issue_gathers(ot)
                drain_gathers(me)
                @pl.when(k >= 2)
                def _():
                    pltpu.make_async_copy(
                        me[4], g_h.at[pl.ds(0, _CHC)], me[7]).wait()

                rowsa, rowsb, outg = me[2], me[3], me[4]

                def edge(e, c3):
                    for j in range(HID // 32):
                        sl_ = pl.ds(j * 16, 16)
                        a = plsc.bitcast(rowsa[e, sl_], jnp.bfloat16)
                        b_ = plsc.bitcast(rowsb[e, sl_], jnp.bfloat16)
                        outg[e, sl_] = plsc.bitcast(a + b_, jnp.int32)
                    return c3
                lax.fori_loop(0, _CHC, edge, 0)
                off = pl.multiple_of(base + k * _CHC, 8)
                pltpu.async_copy(outg, g_h.at[pl.ds(off, _CHC)], me[7])

                @pl.when(k + 2 < _NCC)
                def _():
                    issue_idx(k + 2, me)
        return carry
    lax.fori_loop(0, (_NCC + 1) // 2, chunkpair, 0)
    for sl in slots:
        pltpu.make_async_copy(sl[4], g_h.at[pl.ds(0, _CHC)], sl[7]).wait()


_cls_call = pl.kernel(
    _sc_cls,
    out_type=jax.ShapeDtypeStruct((E, HID // 2), jnp.int32),
    mesh=_MESH,
    compiler_params=_SC_PARAMS,
    scratch_types=[
        pltpu.VMEM((_CHC,), jnp.int32),
        pltpu.VMEM((_CHC,), jnp.int32),
        pltpu.VMEM((_CHC,), jnp.int32),
        pltpu.VMEM((_CHC,), jnp.int32),
        pltpu.VMEM((_CHC, HID // 2), jnp.int32),
        pltpu.VMEM((_CHC, HID // 2), jnp.int32),
        pltpu.VMEM((_CHC, HID // 2), jnp.int32),
        pltpu.VMEM((_CHC, HID // 2), jnp.int32),
        pltpu.VMEM((_CHC, HID // 2), jnp.int32),
        pltpu.VMEM((_CHC, HID // 2), jnp.int32),
        pltpu.SemaphoreType.DMA,
        pltpu.SemaphoreType.DMA,
        pltpu.SemaphoreType.DMA,
        pltpu.SemaphoreType.DMA,
        pltpu.SemaphoreType.DMA,
        pltpu.SemaphoreType.DMA,
    ],
)

_R = 400


def _tc_l0(p_ref, d_ref, x_ref, wl_ref, bl_ref, wr_ref, oa_ref, ob_ref):
    deg = jnp.sum(d_ref[...], axis=1)
    inv = 1.0 / jnp.maximum(deg, 1.0)
    mean = (p_ref[0] + p_ref[1]) * inv[:, None]
    h = mean @ wl_ref[...] + bl_ref[...] + x_ref[...] @ wr_ref[...]
    h = jnp.maximum(h, 0.0)
    oa_ref[...] = h[:, :IN]
    ob_ref[...] = h[:, IN:]


def _tc_l0_call(p, degp, x, Wl0, bl0, Wr0):
    return pl.pallas_call(
        _tc_l0,
        grid=(N // _R,),
        in_specs=[
            pl.BlockSpec((NC, _R, IN), lambda i: (0, i, 0)),
            pl.BlockSpec((_R, NW), lambda i: (i, 0)),
            pl.BlockSpec((_R, IN), lambda i: (i, 0)),
            pl.BlockSpec((IN, HID), lambda i: (0, 0)),
            pl.BlockSpec((1, HID), lambda i: (0, 0)),
            pl.BlockSpec((IN, HID), lambda i: (0, 0)),
        ],
        out_specs=[
            pl.BlockSpec((_R, IN), lambda i: (i, 0)),
            pl.BlockSpec((_R, IN), lambda i: (i, 0)),
        ],
        out_shape=[
            jax.ShapeDtypeStruct((N, IN), jnp.float32),
            jax.ShapeDtypeStruct((N, IN), jnp.float32),
        ],
    )(p, degp, x, Wl0, bl0.reshape(1, HID), Wr0)


def _tc_l1(agg_ref, d_ref, ha_ref, hb_ref, wl_ref, bl_ref, wr_ref,
           wa_ref, wb_ref, a_ref, b_ref):
    deg = jnp.sum(d_ref[...], axis=1)
    inv = 1.0 / jnp.maximum(deg, 1.0)
    aggc = jnp.concatenate([agg_ref[0], agg_ref[1]], axis=-1)
    mean = aggc * inv[:, None]
    h1c = jnp.concatenate([ha_ref[...], hb_ref[...]], axis=-1)
    h2 = mean @ wl_ref[...] + bl_ref[...] + h1c @ wr_ref[...]
    h2 = jnp.maximum(h2, 0.0)
    av = h2 @ wa_ref[...]
    bv = h2 @ wb_ref[...]
    a_ref[...] = _pack_pairs(av[:, :HID // 2], av[:, HID // 2:])
    b_ref[...] = _pack_pairs(bv[:, :HID // 2], bv[:, HID // 2:])


def _tc_l1_call(agg1, degp, h1a, h1b, Wl1, bl1, Wr1, W1a, W1b):
    return pl.pallas_call(
        _tc_l1,
        grid=(N // _R,),
        in_specs=[
            pl.BlockSpec((NC, _R, IN), lambda i: (0, i, 0)),
            pl.BlockSpec((_R, NW), lambda i: (i, 0)),
            pl.BlockSpec((_R, IN), lambda i: (i, 0)),
            pl.BlockSpec((_R, IN), lambda i: (i, 0)),
            pl.BlockSpec((HID, HID), lambda i: (0, 0)),
            pl.BlockSpec((1, HID), lambda i: (0, 0)),
            pl.BlockSpec((HID, HID), lambda i: (0, 0)),
            pl.BlockSpec((HID, HID), lambda i: (0, 0)),
            pl.BlockSpec((HID, HID), lambda i: (0, 0)),
        ],
        out_specs=[
            pl.BlockSpec((_R, HID // 2), lambda i: (i, 0)),
            pl.BlockSpec((_R, HID // 2), lambda i: (i, 0)),
        ],
        out_shape=[
            jax.ShapeDtypeStruct((N, HID // 2), jnp.int32),
            jax.ShapeDtypeStruct((N, HID // 2), jnp.int32),
        ],
    )(agg1, degp, h1a, h1b, Wl1, bl1.reshape(1, HID), Wr1, W1a, W1b)


_ER = E // 8
_RC = 400


def _tc_fin(g_ref, f_ref, w_ref, b_ref, m_ref, o_ref):
    y = jnp.dot(f_ref[...], w_ref[...],
                preferred_element_type=jnp.float32) + b_ref[...]
    u = lax.bitcast_convert_type(g_ref[...], jnp.uint32)
    parts = []
    for m in range(8):
        w = u[:, m * (HID // 2):(m + 1) * (HID // 2)]
        lo = lax.bitcast_convert_type(
            (w & 0xFFFF).astype(jnp.uint16), jnp.bfloat16)
        hi = lax.bitcast_convert_type(
            (w >> 16).astype(jnp.uint16), jnp.bfloat16)
        parts += [lo, hi]
    z = jnp.concatenate(parts, axis=-1).astype(jnp.float32)
    o_ref[...] = jnp.maximum(z + y, 0.0) @ m_ref[...]


def _tc_fin_call(gr, featr, W1cB, b1big, M2):
    return pl.pallas_call(
        _tc_fin,
        grid=(_ER // _RC,),
        in_specs=[
            pl.BlockSpec((_RC, 8 * HID // 2), lambda i: (i, 0)),
            pl.BlockSpec((_RC, IN), lambda i: (i, 0)),
            pl.BlockSpec((IN, 8 * HID), lambda i: (0, 0)),
            pl.BlockSpec((1, 8 * HID), lambda i: (0, 0)),
            pl.BlockSpec((8 * HID, 8), lambda i: (0, 0)),
        ],
        out_specs=pl.BlockSpec((_RC, 8), lambda i: (i, 0)),
        out_shape=jax.ShapeDtypeStruct((_ER, 8), jnp.float32),
    )(gr, featr, W1cB, b1big, M2)


def kernel(x, edge_index, feat_batch, Wl0, bl0, Wr0, Wl1, bl1, Wr1, W1, b1, W2, b2):
    src = edge_index[0].astype(jnp.int32)
    dst = edge_index[1].astype(jnp.int32)

    W1a = W1[:HID]
    W1b = W1[HID:2 * HID]
    W1c = W1[2 * HID:]
    zero16 = jnp.zeros((FEAT, HID), jnp.float32)
    W1cB = jnp.concatenate(
        [jnp.concatenate([W1c if i == k else zero16 for k in range(8)], axis=1)
         for i in range(8)], axis=0)
    b1big = jnp.tile(b1, 8).reshape(1, 8 * HID)
    featr = feat_batch.reshape(_ER, IN)
    M2 = (jnp.eye(8, dtype=jnp.float32)[:, None, :]
          * W2.reshape(1, HID, 1)).reshape(8 * HID, 8)

    p0, degp = _agg0_call(src, dst, x)
    degp = degp.T
    h1a, h1b = _tc_l0_call(p0, degp, x, Wl0, bl0, Wr0)
    agg1 = _agg1_call(src, dst, h1a, h1b)
    A, B = _tc_l1_call(agg1, degp, h1a, h1b, Wl1, bl1, Wr1, W1a, W1b)
    G = _cls_call(src, dst, A, B)
    logits8 = _tc_fin_call(G.reshape(_ER, 8 * HID // 2), featr, W1cB,
                           b1big, M2)
    return logits8.reshape(E) + b2[0]


# speedup vs baseline: 1.0227x; 1.0227x over previous
import jax
import jax.numpy as jnp
from jax import lax
from jax.experimental import pallas as pl
from jax.experimental.pallas import tpu as pltpu
from jax.experimental.pallas import tpu_sc as plsc

N = 10000
E = 320000
IN = 128
HID = 256
FEAT = 16

NC = 2
NS = 16
NW = NC * NS

_MESH = plsc.VectorSubcoreMesh(core_axis_name="c", subcore_axis_name="s")


def _pack_pairs(lo_f32, hi_f32):
    lo = lax.bitcast_convert_type(lo_f32.astype(jnp.bfloat16), jnp.uint16)
    hi = lax.bitcast_convert_type(hi_f32.astype(jnp.bfloat16), jnp.uint16)
    word = lo.astype(jnp.uint32) | (hi.astype(jnp.uint32) << 16)
    return lax.bitcast_convert_type(word, jnp.int32)
_SC_PARAMS = pltpu.CompilerParams(needs_layout_passes=False)

_ETA = E // NW
_CHA = 80
_NCA = _ETA // _CHA
_RCH = 80
_NRC = N // _RCH


def _own_rows(sid, body):
    nch = jnp.where(sid < _NRC % NS, _NRC // NS + 1, _NRC // NS)

    def f(k, c):
        r0 = pl.multiple_of((sid + NS * k) * _RCH, 8)
        body(r0)
        return c
    lax.fori_loop(0, nch, f, 0)


_DW = 8


def _sc_agg0(src_h, dst_h, x_h, outp_h, outd_h,
             idx_s0, idx_d0, idx_s1, idx_d1, rows0, rows1, zb, dacc,
             semis0, semis1, semid0, semid1, semr0, semr1, semsc0, semsc1,
             acc):
    cc = lax.axis_index("c")
    sid = lax.axis_index("s")
    wid = sid * NC + cc
    zv = jnp.zeros((16,), jnp.float32)
    ov = jnp.ones((16,), jnp.float32)
    slots = [(idx_s0, idx_d0, rows0, semis0, semid0, semr0, semsc0),
             (idx_s1, idx_d1, rows1, semis1, semid1, semr1, semsc1)]

    def zrow(r, carry):
        for j in range(IN // 16):
            zb[r, pl.ds(j * 16, 16)] = zv
        return carry
    lax.fori_loop(0, _RCH, zrow, 0)

    def zdrow(r, carry):
        dacc[pl.ds(pl.multiple_of(r * 16, 16), 16)] = zv
        return carry
    lax.fori_loop(0, N // 16, zdrow, 0)

    _own_rows(sid, lambda r0: pltpu.sync_copy(zb, acc.at[pl.ds(r0, _RCH)]))
    plsc.subcore_barrier()

    base = cc * (E // NC) + sid * _ETA

    def issue_idx_s(k, sl):
        off = pl.multiple_of(base + k * _CHA, 8)
        pltpu.async_copy(src_h.at[pl.ds(off, _CHA)], sl[0], sl[3])

    def issue_idx_d(k, sl):
        off = pl.multiple_of(base + k * _CHA, 8)
        pltpu.async_copy(dst_h.at[pl.ds(off, _CHA)], sl[1], sl[4])

    def drain_idx_s(sl):
        pltpu.make_async_copy(src_h.at[pl.ds(0, _CHA)], sl[0], sl[3]).wait()

    def drain_idx_d(sl):
        pltpu.make_async_copy(dst_h.at[pl.ds(0, _CHA)], sl[1], sl[4]).wait()

    def issue_gather(sl):
        pltpu.async_copy(x_h.at[sl[0]], sl[2], sl[5])

    def drain_gather(sl):
        pltpu.make_async_copy(x_h.at[pl.ds(0, _CHA)], sl[2], sl[5]).wait()

    def issue_scat(sl):
        pltpu.async_copy(sl[2], acc.at[sl[1]], sl[6], add=True)

    def drain_scat(sl):
        pltpu.make_async_copy(sl[2], acc.at[sl[1]], sl[6]).wait()

    issue_idx_s(0, slots[0])
    issue_idx_s(1, slots[1])
    issue_idx_d(0, slots[0])
    drain_idx_s(slots[0])
    issue_gather(slots[0])

    def chunkpair(i, carry):
        kk = i * 2
        for b in range(2):
            me, ot = slots[b], slots[1 - b]
            k = kk + b

            @pl.when(k < _NCA)
            def _():
                @pl.when(k >= 1)
                def _():
                    drain_scat(ot)

                @pl.when(k + 1 < _NCA)
                def _():
                    drain_idx_s(ot)
                    issue_gather(ot)
                    issue_idx_d(k + 1, ot)
                drain_gather(me)
                drain_idx_d(me)
                issue_scat(me)

                def dcount(i2, c2):
                    iv = me[1][pl.ds(pl.multiple_of(i2 * 16, 16), 16)]
                    plsc.addupdate_scatter(dacc, [iv], ov)
                    return c2
                lax.fori_loop(0, _CHA // 16, dcount, 0)

                @pl.when(k + 2 < _NCA)
                def _():
                    issue_idx_s(k + 2, me)
        return carry
    lax.fori_loop(0, (_NCA + 1) // 2, chunkpair, 0)
    drain_scat(slots[0])
    plsc.subcore_barrier()

    def ochunk(r0):
        pltpu.sync_copy(acc.at[pl.ds(r0, _RCH)], zb)
        pltpu.sync_copy(zb, outp_h.at[cc, pl.ds(r0, _RCH)])
    _own_rows(sid, ochunk)
    pltpu.sync_copy(dacc, outd_h.at[wid])


_agg0_call = pl.kernel(
    _sc_agg0,
    out_type=(jax.ShapeDtypeStruct((NC, N, IN), jnp.float32),
              jax.ShapeDtypeStruct((NW, N), jnp.float32)),
    mesh=_MESH,
    compiler_params=_SC_PARAMS,
    scratch_types=[
        pltpu.VMEM((_CHA,), jnp.int32),
        pltpu.VMEM((_CHA,), jnp.int32),
        pltpu.VMEM((_CHA,), jnp.int32),
        pltpu.VMEM((_CHA,), jnp.int32),
        pltpu.VMEM((_CHA, IN), jnp.float32),
        pltpu.VMEM((_CHA, IN), jnp.float32),
        pltpu.VMEM((_RCH, IN), jnp.float32),
        pltpu.VMEM((N,), jnp.float32),
        pltpu.SemaphoreType.DMA,
        pltpu.SemaphoreType.DMA,
        pltpu.SemaphoreType.DMA,
        pltpu.SemaphoreType.DMA,
        pltpu.SemaphoreType.DMA,
        pltpu.SemaphoreType.DMA,
        pltpu.SemaphoreType.DMA,
        pltpu.SemaphoreType.DMA,
        pltpu.VMEM_SHARED((N, IN), jnp.float32),
    ],
)

_ETB = E // NS
_CHB = 80
_NCB = _ETB // _CHB


def _sc_agg1(src_h, dst_h, h1a_h, h1b_h, out_h,
             idx_s0, idx_d0, idx_s1, idx_d1, rows0, rows1, zb,
             semis0, semis1, semid0, semid1, semr0, semr1, semsc0, semsc1,
             acc):
    cc = lax.axis_index("c")
    sid = lax.axis_index("s")
    zv = jnp.zeros((16,), jnp.float32)
    slots = [(idx_s0, idx_d0, rows0, semis0, semid0, semr0, semsc0),
             (idx_s1, idx_d1, rows1, semis1, semid1, semr1, semsc1)]

    def zrow(r, carry):
        for j in range(IN // 16):
            zb[r, pl.ds(j * 16, 16)] = zv
        return carry
    lax.fori_loop(0, _RCH, zrow, 0)
    _own_rows(sid, lambda r0: pltpu.sync_copy(zb, acc.at[pl.ds(r0, _RCH)]))
    plsc.subcore_barrier()

    base = sid * _ETB

    def issue_idx_s(k, sl):
        off = pl.multiple_of(base + k * _CHB, 8)
        pltpu.async_copy(src_h.at[pl.ds(off, _CHB)], sl[0], sl[3])

    def issue_idx_d(k, sl):
        off = pl.multiple_of(base + k * _CHB, 8)
        pltpu.async_copy(dst_h.at[pl.ds(off, _CHB)], sl[1], sl[4])

    def drain_idx_s(sl):
        pltpu.make_async_copy(src_h.at[pl.ds(0, _CHB)], sl[0], sl[3]).wait()

    def drain_idx_d(sl):
        pltpu.make_async_copy(dst_h.at[pl.ds(0, _CHB)], sl[1], sl[4]).wait()

    def issue_gather(sl):
        @pl.when(cc == 0)
        def _():
            pltpu.async_copy(h1a_h.at[sl[0]], sl[2], sl[5])

        @pl.when(cc == 1)
        def _():
            pltpu.async_copy(h1b_h.at[sl[0]], sl[2], sl[5])

    def drain_gather(sl):
        pltpu.make_async_copy(h1a_h.at[pl.ds(0, _CHB)], sl[2], sl[5]).wait()

    def issue_scat(sl):
        pltpu.async_copy(sl[2], acc.at[sl[1]], sl[6], add=True)

    def drain_scat(sl):
        pltpu.make_async_copy(sl[2], acc.at[sl[1]], sl[6]).wait()

    issue_idx_s(0, slots[0])
    issue_idx_s(1, slots[1])
    issue_idx_d(0, slots[0])
    drain_idx_s(slots[0])
    issue_gather(slots[0])

    def chunkpair(i, carry):
        kk = i * 2
        for b in range(2):
            me, ot = slots[b], slots[1 - b]
            k = kk + b

            @pl.when(k >= 1)
            def _():
                drain_scat(ot)

            @pl.when(k + 1 < _NCB)
            def _():
                drain_idx_s(ot)
                issue_gather(ot)
                issue_idx_d(k + 1, ot)
            drain_gather(me)
            drain_idx_d(me)
            issue_scat(me)

            @pl.when(k + 2 < _NCB)
            def _():
                issue_idx_s(k + 2, me)
        return carry
    lax.fori_loop(0, _NCB // 2, chunkpair, 0)
    drain_scat(slots[1])
    plsc.subcore_barrier()

    def ochunk(r0):
        pltpu.sync_copy(acc.at[pl.ds(r0, _RCH)], zb)
        pltpu.sync_copy(zb, out_h.at[cc, pl.ds(r0, _RCH)])
    _own_rows(sid, ochunk)


_agg1_call = pl.kernel(
    _sc_agg1,
    out_type=jax.ShapeDtypeStruct((NC, N, IN), jnp.float32),
    mesh=_MESH,
    compiler_params=_SC_PARAMS,
    scratch_types=[
        pltpu.VMEM((_CHB,), jnp.int32),
        pltpu.VMEM((_CHB,), jnp.int32),
        pltpu.VMEM((_CHB,), jnp.int32),
        pltpu.VMEM((_CHB,), jnp.int32),
        pltpu.VMEM((_CHB, IN), jnp.float32),
        pltpu.VMEM((_CHB, IN), jnp.float32),
        pltpu.VMEM((_RCH, IN), jnp.float32),
        pltpu.SemaphoreType.DMA,
        pltpu.SemaphoreType.DMA,
        pltpu.SemaphoreType.DMA,
        pltpu.SemaphoreType.DMA,
        pltpu.SemaphoreType.DMA,
        pltpu.SemaphoreType.DMA,
        pltpu.SemaphoreType.DMA,
        pltpu.SemaphoreType.DMA,
        pltpu.VMEM_SHARED((N, IN), jnp.float32),
    ],
)

_ETC = E // NW
_CHC = 80
_NCC = _ETC // _CHC


def _sc_cls(src_h, dst_h, a_h, b_h, g_h,
            idx_s0, idx_d0, idx_s1, idx_d1,
            rowsa0, rowsb0, outg0, rowsa1, rowsb1, outg1,
            semi0, semi1, semg0, semg1, semo0, semo1):
    cc = lax.axis_index("c")
    sid = lax.axis_index("s")
    wid = sid * NC + cc
    base = wid * _ETC
    slots = [(idx_s0, idx_d0, rowsa0, rowsb0, outg0, semi0, semg0, semo0),
             (idx_s1, idx_d1, rowsa1, rowsb1, outg1, semi1, semg1, semo1)]

    def issue_idx(k, sl):
        off = pl.multiple_of(base + k * _CHC, 8)
        pltpu.async_copy(src_h.at[pl.ds(off, _CHC)], sl[0], sl[5])
        pltpu.async_copy(dst_h.at[pl.ds(off, _CHC)], sl[1], sl[5])

    def drain_idx(sl):
        pltpu.make_async_copy(src_h.at[pl.ds(0, _CHC)], sl[0], sl[5]).wait()
        pltpu.make_async_copy(src_h.at[pl.ds(0, _CHC)], sl[1], sl[5]).wait()

    def issue_gathers(sl):
        pltpu.async_copy(a_h.at[sl[0]], sl[2], sl[6])
        pltpu.async_copy(b_h.at[sl[1]], sl[3], sl[6])

    def drain_gathers(sl):
        for r in (sl[2], sl[3]):
            pltpu.make_async_copy(a_h.at[pl.ds(0, _CHC)], r, sl[6]).wait()

    issue_idx(0, slots[0])
    issue_idx(1, slots[1])
    drain_idx(slots[0])
    issue_gathers(slots[0])

    def chunkpair(i, carry):
        kk = i * 2
        for b in range(2):
            me, ot = slots[b], slots[1 - b]
            k = kk + b

            @pl.when(k < _NCC)
            def _():
                @pl.when(k + 1 < _NCC)
                def _():
                    drain_idx(ot)
                    issue_gathers(ot)
                drain_gathers(me)
                @pl.when(k >= 2)
                def _():
                    pltpu.make_async_copy(
                        me[4], g_h.at[pl.ds(0, _CHC)], me[7]).wait()

                rowsa, rowsb, outg = me[2], me[3], me[4]

                def edge(e, c3):
                    for j in range(HID // 32):
                        sl_ = pl.ds(j * 16, 16)
                        a = plsc.bitcast(rowsa[e, sl_], jnp.bfloat16)
                        b_ = plsc.bitcast(rowsb[e, sl_], jnp.bfloat16)
                        outg[e, sl_] = plsc.bitcast(a + b_, jnp.int32)
                    return c3
                lax.fori_loop(0, _CHC, edge, 0)
                off = pl.multiple_of(base + k * _CHC, 8)
                pltpu.async_copy(outg, g_h.at[pl.ds(off, _CHC)], me[7])

                @pl.when(k + 2 < _NCC)
                def _():
                    issue_idx(k + 2, me)
        return carry
    lax.fori_loop(0, (_NCC + 1) // 2, chunkpair, 0)
    for sl in slots:
        pltpu.make_async_copy(sl[4], g_h.at[pl.ds(0, _CHC)], sl[7]).wait()


_cls_call = pl.kernel(
    _sc_cls,
    out_type=jax.ShapeDtypeStruct((E, HID // 2), jnp.int32),
    mesh=_MESH,
    compiler_params=_SC_PARAMS,
    scratch_types=[
        pltpu.VMEM((_CHC,), jnp.int32),
        pltpu.VMEM((_CHC,), jnp.int32),
        pltpu.VMEM((_CHC,), jnp.int32),
        pltpu.VMEM((_CHC,), jnp.int32),
        pltpu.VMEM((_CHC, HID // 2), jnp.int32),
        pltpu.VMEM((_CHC, HID // 2), jnp.int32),
        pltpu.VMEM((_CHC, HID // 2), jnp.int32),
        pltpu.VMEM((_CHC, HID // 2), jnp.int32),
        pltpu.VMEM((_CHC, HID // 2), jnp.int32),
        pltpu.VMEM((_CHC, HID // 2), jnp.int32),
        pltpu.SemaphoreType.DMA,
        pltpu.SemaphoreType.DMA,
        pltpu.SemaphoreType.DMA,
        pltpu.SemaphoreType.DMA,
        pltpu.SemaphoreType.DMA,
        pltpu.SemaphoreType.DMA,
    ],
)

_R = 400


def _tc_l0(p_ref, d_ref, x_ref, wl_ref, bl_ref, wr_ref, oa_ref, ob_ref):
    deg = jnp.sum(d_ref[...], axis=1)
    inv = 1.0 / jnp.maximum(deg, 1.0)
    mean = (p_ref[0] + p_ref[1]) * inv[:, None]
    h = mean @ wl_ref[...] + bl_ref[...] + x_ref[...] @ wr_ref[...]
    h = jnp.maximum(h, 0.0)
    oa_ref[...] = h[:, :IN]
    ob_ref[...] = h[:, IN:]


def _tc_l0_call(p, degp, x, Wl0, bl0, Wr0):
    return pl.pallas_call(
        _tc_l0,
        grid=(N // _R,),
        in_specs=[
            pl.BlockSpec((NC, _R, IN), lambda i: (0, i, 0)),
            pl.BlockSpec((_R, NW), lambda i: (i, 0)),
            pl.BlockSpec((_R, IN), lambda i: (i, 0)),
            pl.BlockSpec((IN, HID), lambda i: (0, 0)),
            pl.BlockSpec((1, HID), lambda i: (0, 0)),
            pl.BlockSpec((IN, HID), lambda i: (0, 0)),
        ],
        out_specs=[
            pl.BlockSpec((_R, IN), lambda i: (i, 0)),
            pl.BlockSpec((_R, IN), lambda i: (i, 0)),
        ],
        out_shape=[
            jax.ShapeDtypeStruct((N, IN), jnp.float32),
            jax.ShapeDtypeStruct((N, IN), jnp.float32),
        ],
    )(p, degp, x, Wl0, bl0.reshape(1, HID), Wr0)


def _tc_l1(agg_ref, d_ref, ha_ref, hb_ref, wl_ref, bl_ref, wr_ref,
           wa_ref, wb_ref, a_ref, b_ref):
    deg = jnp.sum(d_ref[...], axis=1)
    inv = 1.0 / jnp.maximum(deg, 1.0)
    aggc = jnp.concatenate([agg_ref[0], agg_ref[1]], axis=-1)
    mean = aggc * inv[:, None]
    h1c = jnp.concatenate([ha_ref[...], hb_ref[...]], axis=-1)
    h2 = mean @ wl_ref[...] + bl_ref[...] + h1c @ wr_ref[...]
    h2 = jnp.maximum(h2, 0.0)
    av = h2 @ wa_ref[...]
    bv = h2 @ wb_ref[...]
    a_ref[...] = _pack_pairs(av[:, :HID // 2], av[:, HID // 2:])
    b_ref[...] = _pack_pairs(bv[:, :HID // 2], bv[:, HID // 2:])


def _tc_l1_call(agg1, degp, h1a, h1b, Wl1, bl1, Wr1, W1a, W1b):
    return pl.pallas_call(
        _tc_l1,
        grid=(N // _R,),
        in_specs=[
            pl.BlockSpec((NC, _R, IN), lambda i: (0, i, 0)),
            pl.BlockSpec((_R, NW), lambda i: (i, 0)),
            pl.BlockSpec((_R, IN), lambda i: (i, 0)),
            pl.BlockSpec((_R, IN), lambda i: (i, 0)),
            pl.BlockSpec((HID, HID), lambda i: (0, 0)),
            pl.BlockSpec((1, HID), lambda i: (0, 0)),
            pl.BlockSpec((HID, HID), lambda i: (0, 0)),
            pl.BlockSpec((HID, HID), lambda i: (0, 0)),
            pl.BlockSpec((HID, HID), lambda i: (0, 0)),
        ],
        out_specs=[
            pl.BlockSpec((_R, HID // 2), lambda i: (i, 0)),
            pl.BlockSpec((_R, HID // 2), lambda i: (i, 0)),
        ],
        out_shape=[
            jax.ShapeDtypeStruct((N, HID // 2), jnp.int32),
            jax.ShapeDtypeStruct((N, HID // 2), jnp.int32),
        ],
    )(agg1, degp, h1a, h1b, Wl1, bl1.reshape(1, HID), Wr1, W1a, W1b)


_ER = E // 8
_RC = 400


def _tc_fin(g_ref, f_ref, w_ref, b_ref, m_ref, o_ref):
    y = (jnp.dot(f_ref[...], w_ref[...], preferred_element_type=jnp.float32)
         + b_ref[...]).astype(jnp.bfloat16)
    u = lax.bitcast_convert_type(g_ref[...], jnp.uint32)
    parts = []
    for m in range(8):
        w = u[:, m * (HID // 2):(m + 1) * (HID // 2)]
        lo = lax.bitcast_convert_type(
            (w & 0xFFFF).astype(jnp.uint16), jnp.bfloat16)
        hi = lax.bitcast_convert_type(
            (w >> 16).astype(jnp.uint16), jnp.bfloat16)
        parts += [lo, hi]
    z = jnp.concatenate(parts, axis=-1)
    hdn = jnp.maximum(z + y, jnp.bfloat16(0))
    o_ref[...] = jnp.dot(hdn, m_ref[...], preferred_element_type=jnp.float32)


def _tc_fin_call(gr, featr, W1cB, b1big, M2):
    return pl.pallas_call(
        _tc_fin,
        grid=(_ER // _RC,),
        in_specs=[
            pl.BlockSpec((_RC, 8 * HID // 2), lambda i: (i, 0)),
            pl.BlockSpec((_RC, IN), lambda i: (i, 0)),
            pl.BlockSpec((IN, 8 * HID), lambda i: (0, 0)),
            pl.BlockSpec((1, 8 * HID), lambda i: (0, 0)),
            pl.BlockSpec((8 * HID, 8), lambda i: (0, 0)),
        ],
        out_specs=pl.BlockSpec((_RC, 8), lambda i: (i, 0)),
        out_shape=jax.ShapeDtypeStruct((_ER, 8), jnp.float32),
    )(gr, featr, W1cB, b1big, M2)


def kernel(x, edge_index, feat_batch, Wl0, bl0, Wr0, Wl1, bl1, Wr1, W1, b1, W2, b2):
    src = edge_index[0].astype(jnp.int32)
    dst = edge_index[1].astype(jnp.int32)

    W1a = W1[:HID]
    W1b = W1[HID:2 * HID]
    W1c = W1[2 * HID:]
    zero16 = jnp.zeros((FEAT, HID), jnp.float32)
    W1cB = jnp.concatenate(
        [jnp.concatenate([W1c if i == k else zero16 for k in range(8)], axis=1)
         for i in range(8)], axis=0)
    b1big = jnp.tile(b1, 8).reshape(1, 8 * HID)
    featr = feat_batch.reshape(_ER, IN).astype(jnp.bfloat16)
    M2 = (jnp.eye(8, dtype=jnp.float32)[:, None, :]
          * W2.reshape(1, HID, 1)).reshape(8 * HID, 8)
    t8 = jnp.arange(HID // 2, dtype=jnp.int32)
    perm = jnp.concatenate([t8, t8 + HID // 2])
    colperm = (jnp.arange(8, dtype=jnp.int32)[:, None] * HID
               + perm[None, :]).reshape(8 * HID)
    W1cB = W1cB[:, colperm].astype(jnp.bfloat16)
    b1big = b1big[:, colperm]
    M2 = M2[colperm, :].astype(jnp.bfloat16)

    p0, degp = _agg0_call(src, dst, x)
    degp = degp.T
    h1a, h1b = _tc_l0_call(p0, degp, x, Wl0, bl0, Wr0)
    agg1 = _agg1_call(src, dst, h1a, h1b)
    A, B = _tc_l1_call(agg1, degp, h1a, h1b, Wl1, bl1, Wr1, W1a, W1b)
    G = _cls_call(src, dst, A, B)
    logits8 = _tc_fin_call(G.reshape(_ER, 8 * HID // 2), featr, W1cB,
                           b1big, M2)
    return logits8.reshape(E) + b2[0]
